# baseline (speedup 1.0000x reference)
import jax
import jax.numpy as jnp
from jax.experimental import pallas as pl
from jax.experimental.pallas import tpu as pltpu

NUM_EXPERTS = 64
TOP_K = 2
HIDDEN = 1024
FFN = 512
LANES = 2


def _routing_body(x_ref, gate_ref, w_ref):
    x = x_ref[...]
    gate = gate_ref[...]
    logits = jax.lax.dot_general(
        x, gate, (((1,), (1,)), ((), ())), preferred_element_type=jnp.float32
    )
    T, E = logits.shape
    e_iota = jax.lax.broadcasted_iota(jnp.int32, (T, E), 1)

    l1 = jnp.max(logits, axis=-1, keepdims=True)
    i1 = jnp.min(jnp.where(logits == l1, e_iota, E), axis=-1, keepdims=True)
    masked = jnp.where(e_iota == i1, -jnp.inf, logits)
    l2 = jnp.max(masked, axis=-1, keepdims=True)
    i2 = jnp.min(jnp.where(masked == l2, e_iota, E), axis=-1, keepdims=True)
    w1 = 1.0 / (1.0 + jnp.exp(l2 - l1))
    w2 = 1.0 - w1
    w_ref[...] = (jnp.where(e_iota == i1, w1, 0.0)
                  + jnp.where(e_iota == i2, w2, 0.0))


def _moe_body(ids_ref, flags_ref, x_ref, w_ref, wg_ref, wu_ref, wd_ref, out_ref):
    j = pl.program_id(1)

    @pl.when(j == 0)
    def _init():
        out_ref[...] = jnp.zeros_like(out_ref)

    c = pl.program_id(0)

    @pl.when(flags_ref[c, j] > 0)
    def _step():
        x = x_ref[...]
        g = jax.lax.dot_general(
            x, wg_ref[0], (((1,), (1,)), ((), ())),
            preferred_element_type=jnp.float32)
        u = jax.lax.dot_general(
            x, wu_ref[0], (((1,), (1,)), ((), ())),
            preferred_element_type=jnp.float32)
        h = (g * jax.nn.sigmoid(g)) * u
        o = jax.lax.dot_general(
            h, wd_ref[0], (((1,), (1,)), ((), ())),
            preferred_element_type=jnp.float32)
        T, E = w_ref.shape
        e_iota = jax.lax.broadcasted_iota(jnp.int32, (T, E), 1)
        w_col = jnp.sum(
            jnp.where(e_iota == ids_ref[c, j], w_ref[...], 0.0),
            axis=-1, keepdims=True)
        out_ref[0] += o * w_col


def kernel(hidden_states, gate_w, Wg, Wu, Wd):
    B, S, D = hidden_states.shape
    T = B * S
    E = NUM_EXPERTS
    F = FFN
    C = LANES
    x = hidden_states.reshape(T, D)

    w_dense = pl.pallas_call(
        _routing_body,
        out_shape=jax.ShapeDtypeStruct((T, E), jnp.float32),
    )(x, gate_w)

    e = jnp.arange(E, dtype=jnp.int32)
    active = jnp.any(w_dense > 0.0, axis=0)
    key = jnp.where(active, e, e + E)
    rank = jnp.sum((key[:, None] < key[None, :]).astype(jnp.int32), axis=0)
    hit = (rank[:, None] == e[None, :]).astype(jnp.int32)
    perm = jnp.sum(hit * e[:, None], axis=0)
    flags = jnp.sum(hit * active[:, None].astype(jnp.int32), axis=0)
    last_active = jnp.max(jnp.where(active, e, 0))
    ids = jnp.where(flags > 0, perm, last_active)
    slot = (jnp.arange(C, dtype=jnp.int32)[:, None]
            + C * jnp.arange(E // C, dtype=jnp.int32)[None, :])
    ids2 = ids[slot]
    flags2 = flags[slot]
    lane_fill = jax.lax.cummax(jnp.where(flags2 > 0, ids2, -1), axis=1)
    ids2 = jnp.where(flags2 > 0, ids2, jnp.maximum(lane_fill, 0))

    out2 = pl.pallas_call(
        _moe_body,
        grid_spec=pltpu.PrefetchScalarGridSpec(
            num_scalar_prefetch=2,
            grid=(C, E // C),
            in_specs=[
                pl.BlockSpec((T, D), lambda c, j, ids, flags: (0, 0)),
                pl.BlockSpec((T, E), lambda c, j, ids, flags: (0, 0)),
                pl.BlockSpec((1, F, D), lambda c, j, ids, flags: (ids[c, j], 0, 0)),
                pl.BlockSpec((1, F, D), lambda c, j, ids, flags: (ids[c, j], 0, 0)),
                pl.BlockSpec((1, D, F), lambda c, j, ids, flags: (ids[c, j], 0, 0)),
            ],
            out_specs=pl.BlockSpec((1, T, D), lambda c, j, ids, flags: (c, 0, 0)),
        ),
        out_shape=jax.ShapeDtypeStruct((C, T, D), jnp.float32),
        compiler_params=pltpu.CompilerParams(
            dimension_semantics=("parallel", "arbitrary"),
        ),
    )(ids2, flags2, x, w_dense, Wg, Wu, Wd)

    return jnp.sum(out2, axis=0).reshape(B, S, D)


# speedup vs baseline: 1.0761x; 1.0761x over previous
import jax
import jax.numpy as jnp
from jax.experimental import pallas as pl
from jax.experimental.pallas import tpu as pltpu

NUM_EXPERTS = 64
TOP_K = 2
HIDDEN = 1024
FFN = 512


def _routing_body(x_ref, gate_ref, w_ref, ids_ref, flags_ref):
    x = x_ref[...]
    gate = gate_ref[...]
    logits = jax.lax.dot_general(
        x, gate, (((1,), (1,)), ((), ())), preferred_element_type=jnp.float32
    )
    T, E = logits.shape
    e_iota = jax.lax.broadcasted_iota(jnp.int32, (T, E), 1)

    l1 = jnp.max(logits, axis=-1, keepdims=True)
    i1 = jnp.min(jnp.where(logits == l1, e_iota, E), axis=-1, keepdims=True)
    masked = jnp.where(e_iota == i1, -jnp.inf, logits)
    l2 = jnp.max(masked, axis=-1, keepdims=True)
    i2 = jnp.min(jnp.where(masked == l2, e_iota, E), axis=-1, keepdims=True)
    w1 = 1.0 / (1.0 + jnp.exp(l2 - l1))
    w2 = 1.0 - w1
    w_dense = (jnp.where(e_iota == i1, w1, 0.0)
               + jnp.where(e_iota == i2, w2, 0.0))
    w_ref[...] = w_dense
    active_row = jnp.sum((w_dense > 0.0).astype(jnp.int32),
                         axis=0, keepdims=True) > 0

    logits_t = jax.lax.dot_general(
        gate, x, (((1,), (1,)), ((), ())), preferred_element_type=jnp.float32
    )
    et_iota = jax.lax.broadcasted_iota(jnp.int32, (E, T), 0)
    l1c = jnp.max(logits_t, axis=0, keepdims=True)
    i1c = jnp.min(jnp.where(logits_t == l1c, et_iota, E), axis=0, keepdims=True)
    masked_c = jnp.where(et_iota == i1c, -jnp.inf, logits_t)
    l2c = jnp.max(masked_c, axis=0, keepdims=True)
    i2c = jnp.min(jnp.where(masked_c == l2c, et_iota, E), axis=0, keepdims=True)
    routed_t = (et_iota == i1c) | (et_iota == i2c)
    active_col = jnp.sum(routed_t.astype(jnp.int32),
                         axis=1, keepdims=True) > 0

    e_row = jax.lax.broadcasted_iota(jnp.int32, (1, E), 1)
    e_col = jax.lax.broadcasted_iota(jnp.int32, (E, 1), 0)
    key_row = jnp.where(active_row, e_row, e_row + E)
    key_col = jnp.where(active_col, e_col, e_col + E)
    rank_col = jnp.sum((key_col > key_row).astype(jnp.int32),
                       axis=1, keepdims=True)
    hit = (rank_col == e_row).astype(jnp.int32)
    perm = jnp.sum(hit * e_col, axis=0, keepdims=True)
    flags = jnp.sum(hit * active_col.astype(jnp.int32),
                    axis=0, keepdims=True)
    last_active = jnp.max(jnp.where(active_row, e_row, 0),
                          axis=1, keepdims=True)
    ids_ref[...] = jnp.where(flags > 0, perm, last_active)
    flags_ref[...] = flags


def _moe_body(ids_ref, flags_ref, x_ref, w_ref, wg_ref, wu_ref, wd_ref, out_ref):
    i = pl.program_id(0)

    @pl.when(i == 0)
    def _init():
        out_ref[...] = jnp.zeros_like(out_ref)

    @pl.when(flags_ref[i] > 0)
    def _step():
        x = x_ref[...]
        g = jax.lax.dot_general(
            x, wg_ref[0], (((1,), (1,)), ((), ())),
            preferred_element_type=jnp.float32)
        u = jax.lax.dot_general(
            x, wu_ref[0], (((1,), (1,)), ((), ())),
            preferred_element_type=jnp.float32)
        h = (g * jax.nn.sigmoid(g)) * u
        o = jax.lax.dot_general(
            h, wd_ref[0], (((1,), (1,)), ((), ())),
            preferred_element_type=jnp.float32)
        T, E = w_ref.shape
        e_iota = jax.lax.broadcasted_iota(jnp.int32, (T, E), 1)
        w_col = jnp.sum(
            jnp.where(e_iota == ids_ref[i], w_ref[...], 0.0),
            axis=-1, keepdims=True)
        out_ref[...] += o * w_col


def kernel(hidden_states, gate_w, Wg, Wu, Wd):
    B, S, D = hidden_states.shape
    T = B * S
    E = NUM_EXPERTS
    F = FFN
    x = hidden_states.reshape(T, D)

    w_dense, ids, flags = pl.pallas_call(
        _routing_body,
        out_shape=[
            jax.ShapeDtypeStruct((T, E), jnp.float32),
            jax.ShapeDtypeStruct((1, E), jnp.int32),
            jax.ShapeDtypeStruct((1, E), jnp.int32),
        ],
    )(x, gate_w)
    ids = ids.reshape(E)
    flags = flags.reshape(E)

    out = pl.pallas_call(
        _moe_body,
        grid_spec=pltpu.PrefetchScalarGridSpec(
            num_scalar_prefetch=2,
            grid=(E,),
            in_specs=[
                pl.BlockSpec((T, D), lambda i, ids, flags: (0, 0)),
                pl.BlockSpec((T, E), lambda i, ids, flags: (0, 0)),
                pl.BlockSpec((1, F, D), lambda i, ids, flags: (ids[i], 0, 0)),
                pl.BlockSpec((1, F, D), lambda i, ids, flags: (ids[i], 0, 0)),
                pl.BlockSpec((1, D, F), lambda i, ids, flags: (ids[i], 0, 0)),
            ],
            out_specs=pl.BlockSpec((T, D), lambda i, ids, flags: (0, 0)),
        ),
        out_shape=jax.ShapeDtypeStruct((T, D), jnp.float32),
    )(ids, flags, x, w_dense, Wg, Wu, Wd)

    return out.reshape(B, S, D)
